# Initial kernel scaffold; baseline (speedup 1.0000x reference)
import jax
import jax.numpy as jnp
from jax.experimental import pallas as pl


def kernel(indices, weight):
    raise NotImplementedError("write your pallas kernel here")


# speedup vs baseline: 1.1132x; 1.1132x over previous
import functools

import jax
import jax.numpy as jnp
from jax import lax
from jax.experimental import pallas as pl
from jax.experimental.pallas import tpu as pltpu
from jax.experimental.pallas import tpu_sc as plsc


def _sc_gather(table, idx_flat, total_rows, dim):
    info = plsc.get_sparse_core_info()
    nc, ns = info.num_cores, info.num_subcores
    nw = nc * ns
    rows_per_worker = total_rows // nw
    chunk = 1600
    n_chunks = rows_per_worker // chunk
    mesh = plsc.VectorSubcoreMesh(core_axis_name="c", subcore_axis_name="s")

    @functools.partial(
        pl.kernel,
        mesh=mesh,
        compiler_params=pltpu.CompilerParams(use_tc_tiling_on_sc=False),
        out_type=jax.ShapeDtypeStruct((total_rows, dim), jnp.float32),
        scratch_types=[
            pltpu.VMEM((2, chunk), jnp.int32),
            pltpu.VMEM((2, chunk, dim), jnp.float32),
            pltpu.SemaphoreType.DMA((2,)),
            pltpu.SemaphoreType.DMA((2,)),
            pltpu.SemaphoreType.DMA((2,)),
        ],
    )
    def k(table_hbm, idx_hbm, out_hbm, idx_v, rows_v, idx_sem, gat_sem, out_sem):
        wid = lax.axis_index("s") * nc + lax.axis_index("c")
        base = wid * rows_per_worker

        def start_idx_load(i, slot):
            return pltpu.async_copy(
                idx_hbm.at[pl.ds(base + i * chunk, chunk)], idx_v.at[slot],
                idx_sem.at[slot])

        def start_gather(slot):
            pltpu.async_copy(table_hbm.at[idx_v.at[slot]], rows_v.at[slot],
                             gat_sem.at[slot])

        def start_store(i, slot):
            pltpu.async_copy(
                rows_v.at[slot], out_hbm.at[pl.ds(base + i * chunk, chunk)],
                out_sem.at[slot])

        def wait_gather(slot):
            pltpu.make_async_copy(
                table_hbm.at[pl.ds(0, chunk)], rows_v.at[slot],
                gat_sem.at[slot]).wait()

        def wait_store(slot):
            pltpu.make_async_copy(
                rows_v.at[slot], out_hbm.at[pl.ds(0, chunk)],
                out_sem.at[slot]).wait()

        start_idx_load(0, 0).wait()
        start_gather(0)
        start_idx_load(1, 1).wait()
        start_gather(1)

        def body(i, carry):
            slot = lax.rem(i, 2)
            wait_gather(slot)
            start_store(i, slot)
            start_idx_load(i + 2, slot).wait()
            wait_store(slot)
            start_gather(slot)
            return carry

        lax.fori_loop(0, n_chunks - 2, body, 0)

        def tail(i):
            slot = lax.rem(i, 2)
            wait_gather(slot)
            start_store(i, slot)
            wait_store(slot)

        tail(n_chunks - 2)
        tail(n_chunks - 1)

    return k(table, idx_flat)


def kernel(indices, weight):
    b, h = indices.shape
    dim = weight.shape[1]
    idx_flat = indices.reshape(-1).astype(jnp.int32)
    out = _sc_gather(weight, idx_flat, b * h, dim)
    return out.reshape(b, h, dim)


# speedup vs baseline: 1.8105x; 1.6263x over previous
import functools

import jax
import jax.numpy as jnp
from jax import lax
from jax.experimental import pallas as pl
from jax.experimental.pallas import tpu as pltpu
from jax.experimental.pallas import tpu_sc as plsc


def _sc_gather(table, idx_flat, batch, hist, dim):
    info = plsc.get_sparse_core_info()
    nc, ns = info.num_cores, info.num_subcores
    nw = nc * ns
    b_per_w = batch // nw
    bchunk = 32
    chunk = bchunk * hist
    n_chunks = b_per_w // bchunk
    mesh = plsc.VectorSubcoreMesh(core_axis_name="c", subcore_axis_name="s")

    @functools.partial(
        pl.kernel,
        mesh=mesh,
        compiler_params=pltpu.CompilerParams(use_tc_tiling_on_sc=False),
        out_type=jax.ShapeDtypeStruct((batch, hist, dim), jnp.float32),
        scratch_types=[
            pltpu.VMEM((2, chunk), jnp.int32),
            pltpu.VMEM((2, chunk, dim), jnp.float32),
            pltpu.SemaphoreType.DMA((2,)),
            pltpu.SemaphoreType.DMA((2,)),
            pltpu.SemaphoreType.DMA((2,)),
        ],
    )
    def k(table_hbm, idx_hbm, out_hbm, idx_v, rows_v, idx_sem, gat_sem, out_sem):
        wid = lax.axis_index("s") * nc + lax.axis_index("c")
        base = wid * b_per_w

        def start_idx_load(i, slot):
            return pltpu.async_copy(
                idx_hbm.at[pl.ds((base + i * bchunk) * hist, chunk)],
                idx_v.at[slot], idx_sem.at[slot])

        def start_gather(slot):
            pltpu.async_copy(table_hbm.at[idx_v.at[slot]], rows_v.at[slot],
                             gat_sem.at[slot])

        def start_store(i, slot):
            b0 = base + i * bchunk
            for j in range(bchunk):
                pltpu.async_copy(
                    rows_v.at[slot, pl.ds(j * hist, hist)],
                    out_hbm.at[b0 + j], out_sem.at[slot])

        def wait_gather(slot):
            pltpu.make_async_copy(
                table_hbm.at[pl.ds(0, chunk)], rows_v.at[slot],
                gat_sem.at[slot]).wait()

        def wait_store(slot):
            pltpu.make_async_copy(
                table_hbm.at[pl.ds(0, chunk)], rows_v.at[slot],
                out_sem.at[slot]).wait()

        start_idx_load(0, 0).wait()
        start_gather(0)
        start_idx_load(1, 1).wait()
        start_gather(1)

        def body(i, carry):
            slot = lax.rem(i, 2)
            wait_gather(slot)
            start_store(i, slot)
            start_idx_load(i + 2, slot).wait()
            wait_store(slot)
            start_gather(slot)
            return carry

        lax.fori_loop(0, n_chunks - 2, body, 0)

        def tail(i):
            slot = lax.rem(i, 2)
            wait_gather(slot)
            start_store(i, slot)
            wait_store(slot)

        tail(n_chunks - 2)
        tail(n_chunks - 1)

    return k(table, idx_flat)


def kernel(indices, weight):
    b, h = indices.shape
    dim = weight.shape[1]
    idx_flat = indices.reshape(-1).astype(jnp.int32)
    return _sc_gather(weight, idx_flat, b, h, dim)
